# baseline (speedup 1.0000x reference)
import jax
import jax.numpy as jnp
from jax.experimental import pallas as pl
from jax.experimental.pallas import tpu as pltpu

_PATCH = 16
_MASK_PROB = 0.5
_GRID = 8
_ROWS = 512 // _GRID
_PR = _ROWS // _PATCH


def _mask_body(m_ref, x_ref, o_ref):
    for j in range(_PR):
        m = m_ref[0, j : j + 1, :]
        xs = x_ref[j * _PATCH : (j + 1) * _PATCH, :]
        o_ref[j * _PATCH : (j + 1) * _PATCH, :] = jnp.where(m != 0.0, 0.0, xs)


def kernel(x):
    img = x[0]
    H, W = img.shape
    nH, nW = H // _PATCH, W // _PATCH
    mkey = jax.random.key(12345)
    patch_mask = jax.random.uniform(mkey, (nH, nW)) < _MASK_PROB
    mask_cols = jnp.repeat(patch_mask, _PATCH, axis=1).astype(jnp.float32)
    mask_cols = mask_cols.reshape(_GRID, _PR, W)

    out = pl.pallas_call(
        _mask_body,
        grid=(_GRID,),
        in_specs=[
            pl.BlockSpec((1, _PR, W), lambda i: (i, 0, 0)),
            pl.BlockSpec((_ROWS, W), lambda i: (i, 0)),
        ],
        out_specs=pl.BlockSpec((_ROWS, W), lambda i: (i, 0)),
        out_shape=jax.ShapeDtypeStruct((H, W), img.dtype),
        compiler_params=pltpu.CompilerParams(
            dimension_semantics=("arbitrary",),
        ),
    )(mask_cols, img)
    return out[None]


# speedup vs baseline: 1.8855x; 1.8855x over previous
import jax
import jax.numpy as jnp
from jax.experimental import pallas as pl
from jax.experimental.pallas import tpu as pltpu

_PATCH = 16
_MASK_PROB = 0.5
_GRID = 2
_ROWS = 512 // _GRID
_PR = _ROWS // _PATCH


def _mask_body(m_ref, x_ref, o_ref):
    for j in range(_PR):
        m = m_ref[0, j : j + 1, :]
        xs = x_ref[j * _PATCH : (j + 1) * _PATCH, :]
        o_ref[j * _PATCH : (j + 1) * _PATCH, :] = jnp.where(m != 0.0, 0.0, xs)


def kernel(x):
    img = x[0]
    H, W = img.shape
    nH, nW = H // _PATCH, W // _PATCH
    mkey = jax.random.key(12345)
    patch_mask = jax.random.uniform(mkey, (nH, nW)) < _MASK_PROB
    mask_cols = jnp.repeat(patch_mask, _PATCH, axis=1).astype(jnp.float32)
    mask_cols = mask_cols.reshape(_GRID, _PR, W)

    out = pl.pallas_call(
        _mask_body,
        grid=(_GRID,),
        in_specs=[
            pl.BlockSpec((1, _PR, W), lambda i: (i, 0, 0)),
            pl.BlockSpec((_ROWS, W), lambda i: (i, 0)),
        ],
        out_specs=pl.BlockSpec((_ROWS, W), lambda i: (i, 0)),
        out_shape=jax.ShapeDtypeStruct((H, W), img.dtype),
        compiler_params=pltpu.CompilerParams(
            dimension_semantics=("arbitrary",),
        ),
    )(mask_cols, img)
    return out[None]
